# Initial kernel scaffold; baseline (speedup 1.0000x reference)
import jax
import jax.numpy as jnp
from jax.experimental import pallas as pl


def kernel(x, table):
    raise NotImplementedError("write your pallas kernel here")


# speedup vs baseline: 1.5577x; 1.5577x over previous
import functools

import numpy as np
import jax
import jax.numpy as jnp
from jax import lax
from jax.experimental import pallas as pl
from jax.experimental.pallas import tpu as pltpu
from jax.experimental.pallas import tpu_sc as plsc

VOCAB = 28996
EMB = 512
SEQ = 8192
BASE_FREQ = 1e-05

NUM_CORES = 2
NUM_SUBCORES = 16
NW = NUM_CORES * NUM_SUBCORES
ROWS_PER_W = SEQ // NW
CHUNK = 64
NCHUNK = ROWS_PER_W // CHUNK
LANES = 16


def _pe_host() -> np.ndarray:
    pos = np.arange(EMB, dtype=np.float32)[None, :]
    mult = np.float32(BASE_FREQ) ** np.linspace(0.0, 2.0, SEQ, dtype=np.float32)[:, None]
    return np.sin(pos * mult).astype(np.float32)


_PE = _pe_host()


@functools.partial(
    pl.kernel,
    mesh=plsc.VectorSubcoreMesh(core_axis_name="c", subcore_axis_name="s"),
    out_type=jax.ShapeDtypeStruct((SEQ, EMB), jnp.float32),
    scratch_types=[
        pltpu.VMEM((ROWS_PER_W,), jnp.int32),
        pltpu.VMEM((CHUNK, EMB), jnp.float32),
        pltpu.VMEM((CHUNK, EMB), jnp.float32),
        pltpu.SemaphoreType.DMA,
    ],
)
def _encode(table_hbm, x_hbm, pe_hbm, out_hbm, idx_v, rows_v, pe_v, sem):
    wid = lax.axis_index("s") * NUM_CORES + lax.axis_index("c")
    base = wid * ROWS_PER_W
    pltpu.sync_copy(x_hbm.at[pl.ds(base, ROWS_PER_W)], idx_v)
    for k in range(NCHUNK):
        row0 = base + k * CHUNK
        idx_slice = idx_v.at[pl.ds(k * CHUNK, CHUNK)]
        pltpu.async_copy(table_hbm.at[idx_slice], rows_v, sem).wait()
        pltpu.sync_copy(pe_hbm.at[pl.ds(row0, CHUNK)], pe_v)

        def body(r, _):
            for j in range(EMB // LANES):
                s = pl.ds(j * LANES, LANES)
                rows_v[r, s] = rows_v[r, s] + pe_v[r, s]
            return 0

        lax.fori_loop(0, CHUNK, body, 0)
        pltpu.sync_copy(rows_v, out_hbm.at[pl.ds(row0, CHUNK)])


def kernel(x, table):
    pe = jnp.asarray(_PE)
    return _encode(table, x, pe)


# speedup vs baseline: 1.7813x; 1.1435x over previous
import functools

import numpy as np
import jax
import jax.numpy as jnp
from jax import lax
from jax.experimental import pallas as pl
from jax.experimental.pallas import tpu as pltpu
from jax.experimental.pallas import tpu_sc as plsc

VOCAB = 28996
EMB = 512
SEQ = 8192
BASE_FREQ = 1e-05

NUM_CORES = 2
NUM_SUBCORES = 16
NW = NUM_CORES * NUM_SUBCORES
ROWS_PER_W = SEQ // NW
CHUNK = 32
NCHUNK = ROWS_PER_W // CHUNK
NBUF = 3
LANES = 16


def _pe_host() -> np.ndarray:
    pos = np.arange(EMB, dtype=np.float32)[None, :]
    mult = np.float32(BASE_FREQ) ** np.linspace(0.0, 2.0, SEQ, dtype=np.float32)[:, None]
    return np.sin(pos * mult).astype(np.float32)


_PE = _pe_host()


@functools.partial(
    pl.kernel,
    mesh=plsc.VectorSubcoreMesh(core_axis_name="c", subcore_axis_name="s"),
    out_type=jax.ShapeDtypeStruct((SEQ, EMB), jnp.float32),
    scratch_types=[
        pltpu.VMEM((ROWS_PER_W,), jnp.int32),
        pltpu.VMEM((NBUF, CHUNK, EMB), jnp.float32),
        pltpu.VMEM((NBUF, CHUNK, EMB), jnp.float32),
    ] + [pltpu.SemaphoreType.DMA] * (3 * NBUF),
)
def _encode(table_hbm, x_hbm, pe_hbm, out_hbm, idx_v, rows_v, pe_v, *sems):
    sem_g = sems[0:NBUF]
    sem_pe = sems[NBUF:2 * NBUF]
    sem_out = sems[2 * NBUF:3 * NBUF]
    wid = lax.axis_index("s") * NUM_CORES + lax.axis_index("c")
    base = wid * ROWS_PER_W
    pltpu.sync_copy(x_hbm.at[pl.ds(base, ROWS_PER_W)], idx_v)

    def fetch_start(k):
        b = k % NBUF
        g = pltpu.async_copy(
            table_hbm.at[idx_v.at[pl.ds(k * CHUNK, CHUNK)]], rows_v.at[b],
            sem_g[b])
        p = pltpu.async_copy(
            pe_hbm.at[pl.ds(base + k * CHUNK, CHUNK)], pe_v.at[b], sem_pe[b])
        return g, p

    pending = {k: fetch_start(k) for k in range(min(2, NCHUNK))}
    pending_out = {}
    for k in range(NCHUNK):
        b = k % NBUF
        if k + 2 < NCHUNK:
            if k >= 1:
                pending_out.pop(k - 1).wait()
            pending[k + 2] = fetch_start(k + 2)
        g, p = pending.pop(k)
        g.wait()
        p.wait()

        def body(r, _):
            for j in range(EMB // LANES):
                s = pl.ds(j * LANES, LANES)
                rows_v[b, r, s] = rows_v[b, r, s] + pe_v[b, r, s]
            return 0

        lax.fori_loop(0, CHUNK, body, 0)
        pending_out[k] = pltpu.async_copy(
            rows_v.at[b], out_hbm.at[pl.ds(base + k * CHUNK, CHUNK)], sem_out[b])
    for k in sorted(pending_out):
        pending_out[k].wait()


def kernel(x, table):
    pe = jnp.asarray(_PE)
    return _encode(table, x, pe)


# speedup vs baseline: 1.9009x; 1.0672x over previous
import functools

import numpy as np
import jax
import jax.numpy as jnp
from jax import lax
from jax.experimental import pallas as pl
from jax.experimental.pallas import tpu as pltpu
from jax.experimental.pallas import tpu_sc as plsc

VOCAB = 28996
EMB = 512
SEQ = 8192
BASE_FREQ = 1e-05

NUM_CORES = 2
NUM_SUBCORES = 16
NW = NUM_CORES * NUM_SUBCORES
ROWS_PER_W = SEQ // NW
CHUNK = 32
NCHUNK = ROWS_PER_W // CHUNK
NBUF = 3
LANES = 16
NBLK = EMB // LANES
AUX = 3 * LANES


def _aux_host() -> np.ndarray:
    m = np.float64(BASE_FREQ) ** np.linspace(0.0, 2.0, SEQ, dtype=np.float64)
    c = np.arange(2 * LANES, dtype=np.float64)
    seeds = np.sin(c[None, :] * m[:, None])
    c2 = np.broadcast_to(2.0 * np.cos(LANES * m)[:, None], (SEQ, LANES))
    return np.concatenate([seeds, c2], axis=1).astype(np.float32)


_AUX = _aux_host()


@functools.partial(
    pl.kernel,
    mesh=plsc.VectorSubcoreMesh(core_axis_name="c", subcore_axis_name="s"),
    out_type=jax.ShapeDtypeStruct((SEQ, EMB), jnp.float32),
    scratch_types=[
        pltpu.VMEM((ROWS_PER_W,), jnp.int32),
        pltpu.VMEM((NBUF, CHUNK, EMB), jnp.float32),
        pltpu.VMEM((NBUF, CHUNK, AUX), jnp.float32),
    ] + [pltpu.SemaphoreType.DMA] * (3 * NBUF),
)
def _encode(table_hbm, x_hbm, aux_hbm, out_hbm, idx_v, rows_v, aux_v, *sems):
    sem_g = sems[0:NBUF]
    sem_a = sems[NBUF:2 * NBUF]
    sem_out = sems[2 * NBUF:3 * NBUF]
    wid = lax.axis_index("s") * NUM_CORES + lax.axis_index("c")
    base = wid * ROWS_PER_W
    pltpu.sync_copy(x_hbm.at[pl.ds(base, ROWS_PER_W)], idx_v)

    def fetch_start(k):
        b = k % NBUF
        g = pltpu.async_copy(
            table_hbm.at[idx_v.at[pl.ds(k * CHUNK, CHUNK)]], rows_v.at[b],
            sem_g[b])
        a = pltpu.async_copy(
            aux_hbm.at[pl.ds(base + k * CHUNK, CHUNK)], aux_v.at[b], sem_a[b])
        return g, a

    pending = {k: fetch_start(k) for k in range(min(2, NCHUNK))}
    pending_out = {}
    for k in range(NCHUNK):
        b = k % NBUF
        if k + 2 < NCHUNK:
            if k >= 1:
                pending_out.pop(k - 1).wait()
            pending[k + 2] = fetch_start(k + 2)
        g, a = pending.pop(k)
        g.wait()
        a.wait()

        def body(r, _):
            v0 = aux_v[b, r, pl.ds(0, LANES)]
            v1 = aux_v[b, r, pl.ds(LANES, LANES)]
            c2 = aux_v[b, r, pl.ds(2 * LANES, LANES)]
            s0 = pl.ds(0, LANES)
            rows_v[b, r, s0] = rows_v[b, r, s0] + v0
            s1 = pl.ds(LANES, LANES)
            rows_v[b, r, s1] = rows_v[b, r, s1] + v1
            for j in range(2, NBLK):
                vn = c2 * v1 - v0
                s = pl.ds(j * LANES, LANES)
                rows_v[b, r, s] = rows_v[b, r, s] + vn
                v0, v1 = v1, vn
            return 0

        lax.fori_loop(0, CHUNK, body, 0)
        pending_out[k] = pltpu.async_copy(
            rows_v.at[b], out_hbm.at[pl.ds(base + k * CHUNK, CHUNK)], sem_out[b])
    for k in sorted(pending_out):
        pending_out[k].wait()


def kernel(x, table):
    aux = jnp.asarray(_AUX)
    return _encode(table, x, aux)
